# baseline (speedup 1.0000x reference)
import functools

import jax
import jax.numpy as jnp
from jax import lax
from jax.experimental import pallas as pl
from jax.experimental.pallas import tpu as pltpu
from jax.experimental.pallas import tpu_sc as plsc

_FIELD_IDX = (0,) * 7 + (1,) * 7 + (2,) * 6 + (3,) * 6
_NF = 26
_NT = 4
_D = 16
_B = 4096
_ROWS = 260000
_PAIRS = _NF * (_NF - 1) // 2
_PADF = 32
_EIDX = _NT * _PADF
_EROWS = 41

_NC = 2
_NS = 16
_NW = _NC * _NS
_BPW = _B // _NW
_G = 2
_NG = _BPW // _G
_NBUF = 2

_FLAT_BLK = 5200


def _flatten_body(w_ref, o_ref):
    o_ref[...] = w_ref[0]


@functools.cache
def _tc_flatten():
    return pl.pallas_call(
        _flatten_body,
        grid=(_NT, _ROWS // _FLAT_BLK),
        in_specs=[
            pl.BlockSpec((1, _FLAT_BLK, _D), lambda f, b: (f, b, 0)),
        ],
        out_specs=pl.BlockSpec(
            (_FLAT_BLK, _D), lambda f, b: (f * (_ROWS // _FLAT_BLK) + b, 0)
        ),
        out_shape=jax.ShapeDtypeStruct((_NT * _ROWS, _D), jnp.float32),
        compiler_params=pltpu.CompilerParams(dimension_semantics=("arbitrary", "arbitrary")),
    )


def _fused_body(w_hbm, gidx_hbm, out_hbm, idx_v, e_v, out_v, sem_g, sem_w):
    wid = lax.axis_index("s") * _NC + lax.axis_index("c")
    b0 = wid * _BPW

    pltpu.sync_copy(gidx_hbm.at[pl.ds(b0, _BPW)], idx_v)

    def fire_gather(gg, rb):
        for k in range(_G):
            pltpu.async_copy(
                w_hbm.at[idx_v.at[gg * _G + k]], e_v[rb].at[k], sem_g[rb]
            )

    def drain_gather(rb):
        for k in range(_G):
            pltpu.make_async_copy(
                w_hbm.at[pl.ds(0, _EIDX)], e_v[rb].at[k], sem_g[rb]
            ).wait()

    def fire_writeback(gg, rb):
        pltpu.async_copy(
            out_v[rb],
            out_hbm.at[pl.ds((b0 + gg * _G) * _EROWS, _G * _EROWS)],
            sem_w[rb],
        )

    def drain_writeback(rb):
        pltpu.make_async_copy(
            out_v[rb], out_hbm.at[pl.ds(0, _G * _EROWS)], sem_w[rb]
        ).wait()

    def compute(rb):
        for k in range(_G):
            p = 0
            for i in range(_NF - 1):
                fi = _FIELD_IDX[i]
                j = i + 1
                while j < _NF:
                    fj = _FIELD_IDX[j]
                    va = e_v[rb][k, fj * _PADF + i]
                    while j < _NF and _FIELD_IDX[j] == fj:
                        out_v[rb][
                            k * _EROWS + p // 8, pl.ds((p % 8) * _D, _D)
                        ] = va * e_v[rb][k, fi * _PADF + j]
                        p += 1
                        j += 1

    for rb in range(_NBUF):
        fire_gather(rb, rb)

    def body(g):
        for rb in range(_NBUF):
            gg = g + rb
            drain_gather(rb)

            @pl.when(gg >= _NBUF)
            def _():
                drain_writeback(rb)

            compute(rb)
            fire_writeback(gg, rb)

            @pl.when(gg + _NBUF < _NG)
            def _():
                fire_gather(gg + _NBUF, rb)

    pl.loop(0, _NG, step=_NBUF)(body)

    for rb in range(_NBUF):
        drain_writeback(rb)


@functools.cache
def _sc_fused():
    return functools.partial(
        pl.kernel,
        mesh=plsc.VectorSubcoreMesh(core_axis_name="c", subcore_axis_name="s"),
        out_type=jax.ShapeDtypeStruct((_B * _EROWS, 128), jnp.float32),
        scratch_types=[
            pltpu.VMEM((_BPW, _EIDX), jnp.int32),
            [pltpu.VMEM((_G, _EIDX, _D), jnp.float32) for _ in range(_NBUF)],
            [pltpu.VMEM((_G * _EROWS, 128), jnp.float32) for _ in range(_NBUF)],
            [pltpu.SemaphoreType.DMA for _ in range(_NBUF)],
            [pltpu.SemaphoreType.DMA for _ in range(_NBUF)],
        ],
        compiler_params=pltpu.CompilerParams(use_tc_tiling_on_sc=False),
    )(_fused_body)


def kernel(x, W):
    xi = x + (jnp.arange(_NF, dtype=x.dtype) * 10000)[None, :]
    xip = jnp.concatenate(
        [xi, jnp.zeros((_B, _PADF - _NF), dtype=xi.dtype)], axis=1
    )
    gidx = (
        xip[:, None, :]
        + (jnp.arange(_NT, dtype=x.dtype) * _ROWS)[None, :, None]
    ).reshape(_B, _EIDX)

    wf = _tc_flatten()(W)
    out = _sc_fused()(wf, gidx)
    return out.reshape(_B, _EROWS * 128)[:, : _PAIRS * _D].reshape(
        _B, _PAIRS, _D
    )


# speedup vs baseline: 1.7539x; 1.7539x over previous
import functools

import jax
import jax.numpy as jnp
from jax import lax
from jax.experimental import pallas as pl
from jax.experimental.pallas import tpu as pltpu
from jax.experimental.pallas import tpu_sc as plsc

_FIELD_IDX = (0,) * 7 + (1,) * 7 + (2,) * 6 + (3,) * 6
_NF = 26
_NT = 4
_D = 16
_B = 4096
_ROWS = 260000
_PAIRS = _NF * (_NF - 1) // 2
_PADF = 32
_EIDX = _NT * _PADF
_EROWS = 41

_NC = 2
_NS = 16
_NW = _NC * _NS
_BPW = _B // _NW
_G = 2
_NG = _BPW // _G
_NBUF = 2

def _fused_body(w_hbm, gidx_hbm, out_hbm, idx_v, e_v, out_v, sem_g, sem_w):
    wid = lax.axis_index("s") * _NC + lax.axis_index("c")
    b0 = wid * _BPW

    pltpu.sync_copy(gidx_hbm.at[pl.ds(b0, _BPW)], idx_v)

    def fire_gather(gg, rb):
        for k in range(_G):
            for f in range(_NT):
                pltpu.async_copy(
                    w_hbm.at[f].at[
                        idx_v.at[gg * _G + k, pl.ds(f * _PADF, _PADF)]
                    ],
                    e_v[rb].at[k, pl.ds(f * _PADF, _PADF)],
                    sem_g[rb],
                )

    def drain_gather(rb):
        for k in range(_G):
            for f in range(_NT):
                pltpu.make_async_copy(
                    w_hbm.at[0].at[pl.ds(0, _PADF)],
                    e_v[rb].at[k, pl.ds(f * _PADF, _PADF)],
                    sem_g[rb],
                ).wait()

    def fire_writeback(gg, rb):
        pltpu.async_copy(
            out_v[rb],
            out_hbm.at[pl.ds((b0 + gg * _G) * _EROWS, _G * _EROWS)],
            sem_w[rb],
        )

    def drain_writeback(rb):
        pltpu.make_async_copy(
            out_v[rb], out_hbm.at[pl.ds(0, _G * _EROWS)], sem_w[rb]
        ).wait()

    def compute(rb):
        for k in range(_G):
            p = 0
            for i in range(_NF - 1):
                fi = _FIELD_IDX[i]
                j = i + 1
                while j < _NF:
                    fj = _FIELD_IDX[j]
                    va = e_v[rb][k, fj * _PADF + i]
                    while j < _NF and _FIELD_IDX[j] == fj:
                        out_v[rb][
                            k * _EROWS + p // 8, pl.ds((p % 8) * _D, _D)
                        ] = va * e_v[rb][k, fi * _PADF + j]
                        p += 1
                        j += 1

    for rb in range(_NBUF):
        fire_gather(rb, rb)

    def body(g):
        for rb in range(_NBUF):
            gg = g + rb
            drain_gather(rb)

            @pl.when(gg >= _NBUF)
            def _():
                drain_writeback(rb)

            compute(rb)
            fire_writeback(gg, rb)

            @pl.when(gg + _NBUF < _NG)
            def _():
                fire_gather(gg + _NBUF, rb)

    pl.loop(0, _NG, step=_NBUF)(body)

    for rb in range(_NBUF):
        drain_writeback(rb)


@functools.cache
def _sc_fused():
    return functools.partial(
        pl.kernel,
        mesh=plsc.VectorSubcoreMesh(core_axis_name="c", subcore_axis_name="s"),
        out_type=jax.ShapeDtypeStruct((_B * _EROWS, 128), jnp.float32),
        scratch_types=[
            pltpu.VMEM((_BPW, _EIDX), jnp.int32),
            [pltpu.VMEM((_G, _EIDX, _D), jnp.float32) for _ in range(_NBUF)],
            [pltpu.VMEM((_G * _EROWS, 128), jnp.float32) for _ in range(_NBUF)],
            [pltpu.SemaphoreType.DMA for _ in range(_NBUF)],
            [pltpu.SemaphoreType.DMA for _ in range(_NBUF)],
        ],
        compiler_params=pltpu.CompilerParams(use_tc_tiling_on_sc=False),
    )(_fused_body)


def kernel(x, W):
    xi = x + (jnp.arange(_NF, dtype=x.dtype) * 10000)[None, :]
    xip = jnp.concatenate(
        [xi, jnp.zeros((_B, _PADF - _NF), dtype=xi.dtype)], axis=1
    )
    gidx = jnp.tile(xip, (1, _NT))

    out = _sc_fused()(W, gidx)
    return out.reshape(_B, _EROWS * 128)[:, : _PAIRS * _D].reshape(
        _B, _PAIRS, _D
    )


# speedup vs baseline: 1.9706x; 1.1236x over previous
import functools

import jax
import jax.numpy as jnp
from jax import lax
from jax.experimental import pallas as pl
from jax.experimental.pallas import tpu as pltpu
from jax.experimental.pallas import tpu_sc as plsc

_FIELD_IDX = (0,) * 7 + (1,) * 7 + (2,) * 6 + (3,) * 6
_NF = 26
_NT = 4
_D = 16
_B = 4096
_ROWS = 260000
_PAIRS = _NF * (_NF - 1) // 2
_LOOK = _NT * _NF

_NC = 2
_NS = 16
_NW = _NC * _NS
_PER_W = _B * _LOOK // _NW
_IDXROWS = _PER_W // 128
_CHUNKS = 8
_CROWS = _PER_W // _CHUNKS
_G = _CROWS // 128


def _sc_gather_body(w_hbm, gidx_hbm, out_hbm, idx_v, rows_v, sem):
    wid = lax.axis_index("s") * _NC + lax.axis_index("c")
    pltpu.sync_copy(gidx_hbm.at[pl.ds(wid * _IDXROWS, _IDXROWS)], idx_v)
    for c in range(_CHUNKS):
        cps = [
            pltpu.async_copy(
                w_hbm.at[idx_v.at[c * _G + g]],
                rows_v.at[pl.ds(g * 128, 128)],
                sem,
            )
            for g in range(_G)
        ]
        for cp in cps:
            cp.wait()
        pltpu.sync_copy(
            rows_v, out_hbm.at[pl.ds(wid * _PER_W + c * _CROWS, _CROWS)]
        )


@functools.cache
def _sc_gather():
    return functools.partial(
        pl.kernel,
        mesh=plsc.VectorSubcoreMesh(core_axis_name="c", subcore_axis_name="s"),
        out_type=jax.ShapeDtypeStruct((_B * _LOOK, _D), jnp.float32),
        scratch_types=[
            pltpu.VMEM((_IDXROWS, 128), jnp.int32),
            pltpu.VMEM((_CROWS, _D), jnp.float32),
            pltpu.SemaphoreType.DMA,
        ],
        compiler_params=pltpu.CompilerParams(use_tc_tiling_on_sc=False),
    )(_sc_gather_body)


_SEGS = []
for _i in range(_NF - 1):
    _segs = []
    _j = _i + 1
    while _j < _NF:
        _f = _FIELD_IDX[_j]
        _j2 = _j
        while _j2 < _NF and _FIELD_IDX[_j2] == _f:
            _j2 += 1
        _segs.append((_f, _j2 - _j))
        _j = _j2
    _SEGS.append(_segs)


def _pairs_body(e_ref, o_ref):
    e = e_ref[...]
    parts = []
    for i in range(_NF - 1):
        fi = _FIELD_IDX[i]
        r = e[:, (fi * _NF + i + 1) * _D:(fi * _NF + _NF) * _D]
        lsegs = []
        for f, cnt in _SEGS[i]:
            v = e[:, (f * _NF + i) * _D:(f * _NF + i + 1) * _D]
            lsegs.append(v if cnt == 1 else jnp.tile(v, (1, cnt)))
        left = lsegs[0] if len(lsegs) == 1 else jnp.concatenate(lsegs, axis=1)
        parts.append(left * r)
    o_ref[...] = jnp.concatenate(parts, axis=1)


_BB = 256
_TC_PAIRS = pl.pallas_call(
    _pairs_body,
    grid=(_B // _BB,),
    in_specs=[pl.BlockSpec((_BB, _LOOK * _D), lambda i: (i, 0))],
    out_specs=pl.BlockSpec((_BB, _PAIRS * _D), lambda i: (i, 0)),
    out_shape=jax.ShapeDtypeStruct((_B, _PAIRS * _D), jnp.float32),
    compiler_params=pltpu.CompilerParams(dimension_semantics=("arbitrary",)),
)


def kernel(x, W):
    wf = W.reshape(_NT * _ROWS, _D)
    feat_offs = (jnp.arange(_NF, dtype=x.dtype) * 10000)[None, None, :]
    table_offs = (jnp.arange(_NT, dtype=x.dtype) * _ROWS)[None, :, None]
    gidx = (x[:, None, :] + feat_offs + table_offs).reshape(_IDXROWS * _NW, 128)
    e = _sc_gather()(wf, gidx)
    out = _TC_PAIRS(e.reshape(_B, _LOOK * _D))
    return out.reshape(_B, _PAIRS, _D)
